# Initial kernel scaffold; baseline (speedup 1.0000x reference)
import jax
import jax.numpy as jnp
from jax.experimental import pallas as pl


def kernel(x, edge_index, edge_type, params):
    raise NotImplementedError("write your pallas kernel here")


# speedup vs baseline: 12.9602x; 12.9602x over previous
import functools

import jax
import jax.numpy as jnp
from jax import lax
from jax.experimental import pallas as pl
from jax.experimental.pallas import tpu as pltpu
from jax.experimental.pallas import tpu_sc as plsc

R = 8
NB_BASES = 4
N = 10000
E = 320000
D = 128

NC = 2
NS = 16
NW = NC * NS

EW = E // NW
KB = 128
NCH = EW // KB
ZRA = 624
ZRL = N - 15 * ZRA

NKEY = N * R
KPT = NKEY // NS
CH1 = 1600
CH3 = 2000


def _sc_mesh():
    return plsc.VectorSubcoreMesh(core_axis_name="c", subcore_axis_name="s")


NRANGE = 320
NPAD = NW * NRANGE
CAP = 12160
NKL = NRANGE * R
NCHB = CAP // KB


@functools.partial(
    pl.kernel,
    out_type=[
        jax.ShapeDtypeStruct((NW, 1, CAP), jnp.int32),
        jax.ShapeDtypeStruct((NW, 1, CAP), jnp.int32),
        jax.ShapeDtypeStruct((NW, 1, CAP), jnp.float32),
    ],
    mesh=_sc_mesh(),
    scratch_types=[
        pltpu.VMEM((CAP,), jnp.int32),
        pltpu.VMEM((CAP,), jnp.int32),
        pltpu.VMEM((CAP,), jnp.int32),
        pltpu.VMEM((1, CAP), jnp.int32),
        pltpu.VMEM((1, CAP), jnp.int32),
        pltpu.VMEM((1, CAP), jnp.float32),
        pltpu.VMEM((NKL,), jnp.float32),
        pltpu.VMEM((16,), jnp.int32),
    ],
)
def _binedges(src_hbm, dst_hbm, rel_hbm, est_hbm, hrowl_hbm, idxl_hbm,
              norml_hbm, sbuf, dbuf, rbuf, hrowl, idxl, norml, kcnt, ebuf):
    c = lax.axis_index("c")
    s = lax.axis_index("s")
    wid = c * NS + s
    base = wid * NRANGE
    iota = lax.broadcasted_iota(jnp.int32, (16,), 0)

    pltpu.sync_copy(est_hbm.at[wid, 0], ebuf)
    eb = pl.multiple_of(ebuf[pl.ds(0, 16)][0], 8)
    pltpu.sync_copy(src_hbm.at[pl.ds(eb, CAP)], sbuf)
    pltpu.sync_copy(dst_hbm.at[pl.ds(eb, CAP)], dbuf)
    pltpu.sync_copy(rel_hbm.at[pl.ds(eb, CAP)], rbuf)

    def zk(g, _):
        kcnt[pl.ds(g * 16, 16)] = jnp.zeros((16,), jnp.float32)
        return 0
    lax.fori_loop(0, NKL // 16, zk, 0)

    def count_grp(g, _):
        sl = pl.ds(g * 16, 16)
        d16 = dbuf[sl]
        r16 = rbuf[sl]
        m = (d16 >= base) & (d16 < base + NRANGE)
        mi = jnp.where(m, 1, 0)
        kv = jnp.where(m, (d16 - base) * R + r16, 0)
        for jj in range(16):
            @pl.when(mi[jj] > 0)
            def _():
                k = kv[jj]
                kb = (k >> 4) << 4
                lane = k & 15
                oh = jnp.where(iota == lane, 1.0, 0.0)
                kcnt[pl.ds(kb, 16)] = kcnt[pl.ds(kb, 16)] + oh
        return 0
    lax.fori_loop(0, CAP // 16, count_grp, 0)

    def emit_grp(g, _):
        sl = pl.ds(g * 16, 16)
        s16 = sbuf[sl]
        d16 = dbuf[sl]
        r16 = rbuf[sl]
        m = (d16 >= base) & (d16 < base + NRANGE)
        kv = jnp.where(m, (d16 - base) * R + r16, 0)
        hrowl[0, sl] = r16 * N + s16
        idxl[0, sl] = s * NRANGE + jnp.where(m, kv >> 3, 0)
        nv = jnp.zeros((16,), jnp.float32)
        for jj in range(16):
            k = kv[jj]
            kb = (k >> 4) << 4
            lane = k & 15
            row = kcnt[pl.ds(kb, 16)]
            bc = lax.gather(
                row, jnp.full((16, 1), lane, jnp.int32),
                dimension_numbers=lax.GatherDimensionNumbers(
                    offset_dims=(), collapsed_slice_dims=(0,),
                    start_index_map=(0,)),
                slice_sizes=(1,),
                mode=lax.GatherScatterMode.PROMISE_IN_BOUNDS)
            nv = jnp.where(iota == jj, 1.0 / jnp.maximum(bc, 1.0), nv)
        norml[0, sl] = jnp.where(m, nv, 0.0)
        return 0
    lax.fori_loop(0, CAP // 16, emit_grp, 0)

    pltpu.sync_copy(hrowl, hrowl_hbm.at[wid])
    pltpu.sync_copy(idxl, idxl_hbm.at[wid])
    pltpu.sync_copy(norml, norml_hbm.at[wid])


SC_ROWS = NS * NRANGE


@functools.partial(
    pl.kernel,
    out_type=jax.ShapeDtypeStruct((NPAD, D), jnp.float32),
    mesh=_sc_mesh(),
    scratch_types=[
        pltpu.VMEM((1, CAP), jnp.int32),
        pltpu.VMEM((1, CAP), jnp.int32),
        pltpu.VMEM((1, CAP), jnp.float32),
        pltpu.VMEM((KB,), jnp.int32),
        pltpu.VMEM((KB, D), jnp.float32),
        pltpu.VMEM((40, D), jnp.float32),
        pltpu.VMEM_SHARED((SC_ROWS, D), jnp.float32),
        pltpu.SemaphoreType.DMA,
    ],
)
def _scatter(h_hbm, hrowl_hbm, idxl_hbm, norml_hbm, out_hbm,
             hrowl, idxl, norml, dstc, rows_v, zb, acc_sh, sem):
    c = lax.axis_index("c")
    s = lax.axis_index("s")
    wid = c * NS + s

    pltpu.sync_copy(hrowl_hbm.at[wid], hrowl)
    pltpu.sync_copy(idxl_hbm.at[wid], idxl)
    pltpu.sync_copy(norml_hbm.at[wid], norml)

    def zf(j, _):
        for l in range(D // 16):
            zb[j, pl.ds(l * 16, 16)] = jnp.zeros((16,), jnp.float32)
        return 0
    lax.fori_loop(0, 40, zf, 0)

    def zc(k, _):
        pltpu.sync_copy(zb, acc_sh.at[pl.ds(s * NRANGE + k * 40, 40)])
        return 0
    lax.fori_loop(0, NRANGE // 40, zc, 0)

    def chunk(i, _):
        cb = i * KB
        pltpu.async_copy(h_hbm.at[hrowl.at[0, pl.ds(cb, KB)]],
                         rows_v, sem).wait()

        def stage(g, _):
            sl = pl.ds(g * 16, 16)
            dstc[sl] = idxl[0, pl.ds(cb + g * 16, 16)]
            return 0
        lax.fori_loop(0, KB // 16, stage, 0)

        def scale(g, _):
            nv16 = norml[0, pl.ds(cb + g * 16, 16)]
            for jj in range(16):
                j = g * 16 + jj
                nr = nv16[jj]
                for l in range(D // 16):
                    sl = pl.ds(l * 16, 16)
                    rows_v[j, sl] = rows_v[j, sl] * nr
            return 0
        lax.fori_loop(0, KB // 16, scale, 0)
        pltpu.sync_copy(rows_v, acc_sh.at[dstc], add=True)
        return 0
    lax.fori_loop(0, NCHB, chunk, 0)

    pltpu.sync_copy(acc_sh.at[pl.ds(s * NRANGE, NRANGE)],
                    out_hbm.at[pl.ds(wid * NRANGE, NRANGE)])


BN = 2000
NBLK = N // BN


def _dense_a_body(comp_ref, x_ref, bases_ref, root_ref, skw_ref, biasc_ref,
                  h_ref, yd_ref, hb_ref):
    r = pl.program_id(1)

    @pl.when(r == 0)
    def _():
        x = x_ref[...]
        for b in range(NB_BASES):
            hb_ref[b] = jnp.dot(x, bases_ref[b],
                                preferred_element_type=jnp.float32)
        yd_ref[...] = (jnp.dot(x, root_ref[...] + skw_ref[...],
                               preferred_element_type=jnp.float32)
                       + biasc_ref[...])

    acc = comp_ref[r, 0] * hb_ref[0]
    for b in range(1, NB_BASES):
        acc = acc + comp_ref[r, b] * hb_ref[b]
    h_ref[0] = acc


def _dense_a(x, bases, comp, root, skw, biasc):
    return pl.pallas_call(
        _dense_a_body,
        grid=(NBLK, R),
        in_specs=[
            pl.BlockSpec(memory_space=pltpu.SMEM),
            pl.BlockSpec((BN, D), lambda nb, r: (nb, 0)),
            pl.BlockSpec((NB_BASES, D, D), lambda nb, r: (0, 0, 0)),
            pl.BlockSpec((D, D), lambda nb, r: (0, 0)),
            pl.BlockSpec((D, D), lambda nb, r: (0, 0)),
            pl.BlockSpec((1, D), lambda nb, r: (0, 0)),
        ],
        out_specs=[
            pl.BlockSpec((1, BN, D), lambda nb, r: (r, nb, 0)),
            pl.BlockSpec((BN, D), lambda nb, r: (nb, 0)),
        ],
        out_shape=[
            jax.ShapeDtypeStruct((R, N, D), jnp.float32),
            jax.ShapeDtypeStruct((N, D), jnp.float32),
        ],
        scratch_shapes=[pltpu.VMEM((NB_BASES, BN, D), jnp.float32)],
    )(comp, x, bases, root, skw, biasc)


def _c1_body(agg_ref, yd_ref, t_ref, sum_ref, sq_ref):
    t = agg_ref[...] + yd_ref[...]
    t_ref[...] = t

    @pl.when(pl.program_id(0) == 0)
    def _():
        sum_ref[...] = jnp.zeros_like(sum_ref)
        sq_ref[...] = jnp.zeros_like(sq_ref)

    sum_ref[...] += jnp.sum(t, axis=0, keepdims=True)
    sq_ref[...] += jnp.sum(t * t, axis=0, keepdims=True)


def _c1(agg, yd):
    return pl.pallas_call(
        _c1_body,
        grid=(NBLK,),
        in_specs=[
            pl.BlockSpec((BN, D), lambda nb: (nb, 0)),
            pl.BlockSpec((BN, D), lambda nb: (nb, 0)),
        ],
        out_specs=[
            pl.BlockSpec((BN, D), lambda nb: (nb, 0)),
            pl.BlockSpec((1, D), lambda nb: (0, 0)),
            pl.BlockSpec((1, D), lambda nb: (0, 0)),
        ],
        out_shape=[
            jax.ShapeDtypeStruct((N, D), jnp.float32),
            jax.ShapeDtypeStruct((1, D), jnp.float32),
            jax.ShapeDtypeStruct((1, D), jnp.float32),
        ],
    )(agg, yd)


def _c2_body(t_ref, sum_ref, sq_ref, gamma_ref, beta_ref, o_ref):
    inv_n = 1.0 / N
    m = sum_ref[...] * inv_n
    v = sq_ref[...] * inv_n - m * m
    inv = lax.rsqrt(v + 1e-5)
    o_ref[...] = jnp.maximum(
        gamma_ref[...] * (t_ref[...] - m) * inv + beta_ref[...], 0.0)


def _c2(t, ssum, ssq, gamma, beta):
    return pl.pallas_call(
        _c2_body,
        grid=(NBLK,),
        in_specs=[
            pl.BlockSpec((BN, D), lambda nb: (nb, 0)),
            pl.BlockSpec((1, D), lambda nb: (0, 0)),
            pl.BlockSpec((1, D), lambda nb: (0, 0)),
            pl.BlockSpec((1, D), lambda nb: (0, 0)),
            pl.BlockSpec((1, D), lambda nb: (0, 0)),
        ],
        out_specs=pl.BlockSpec((BN, D), lambda nb: (nb, 0)),
        out_shape=jax.ShapeDtypeStruct((N, D), jnp.float32),
    )(t, ssum, ssq, gamma, beta)


def kernel(x, edge_index, edge_type, params):
    src = edge_index[0]
    dst = edge_index[1]
    rel = edge_type

    order = jnp.argsort(dst)
    src_s = src[order]
    dst_s = dst[order]
    rel_s = rel[order]
    estart = jnp.searchsorted(dst_s, jnp.arange(NW) * NRANGE).astype(jnp.int32)
    eal = jnp.minimum((estart // 8) * 8, E - CAP)
    est = jnp.broadcast_to(eal[:, None], (NW, 16)).reshape(NW, 1, 16)
    hrowl, idxl, norml = _binedges(src_s, dst_s, rel_s, est)

    x_cur = x
    hsnap = None
    out = None
    for i in range(3):
        p = params[f'conv{i}']
        sk = params[f'skip{i}']
        biasc = (p['bias'] + sk['b']).reshape(1, D)
        h8, yd = _dense_a(x_cur, p['bases'], p['comp'], p['root'],
                          sk['W'], biasc)
        agg = _scatter(h8.reshape(R * N, D), hrowl, idxl, norml)
        t, ssum, ssq = _c1(agg[:N], yd)
        if i < 2:
            bn = params[f'bn{i}']
            x_cur = _c2(t, ssum, ssq, bn['gamma'].reshape(1, D),
                        bn['beta'].reshape(1, D))
            if i == 1:
                hsnap = t
        else:
            out = t
    return (hsnap, out)
